# baseline (speedup 1.0000x reference)
import jax
import jax.numpy as jnp
from jax.experimental import pallas as pl

CST = 1e-05


def _one_plus_elu(x):
    return jnp.where(x > 0, 1.0 + x, jnp.exp(jnp.minimum(x, 0.0)))


def _dense_body(x_ref, win_ref, wq_ref, wk_ref, wv_ref, h_ref, q_ref, k_ref, v_ref):
    x = x_ref[...]
    h = jax.nn.relu(jnp.dot(x, win_ref[...], preferred_element_type=jnp.float32))
    q = _one_plus_elu(jnp.dot(h, wq_ref[...], preferred_element_type=jnp.float32))
    k = _one_plus_elu(jnp.dot(h, wk_ref[...], preferred_element_type=jnp.float32))
    v = jnp.dot(h, wv_ref[...], preferred_element_type=jnp.float32)
    h_ref[...] = h
    q_ref[...] = q
    k_ref[...] = k
    v_ref[...] = v


def kernel(node_feat, edge_index, W_in, b_in, W_q, b_q, W_k, b_k, W_v, b_v, hopwise, teleport):
    n = node_feat.shape[0]
    C = W_q.shape[0]
    NCLS = W_v.shape[1]
    row = edge_index[0]
    col = edge_index[1]

    grid = (n // 2000,)
    h, Q, Kf, V = pl.pallas_call(
        _dense_body,
        grid=grid,
        in_specs=[
            pl.BlockSpec((2000, node_feat.shape[1]), lambda i: (i, 0)),
            pl.BlockSpec(W_in.shape, lambda i: (0, 0)),
            pl.BlockSpec(W_q.shape, lambda i: (0, 0)),
            pl.BlockSpec(W_k.shape, lambda i: (0, 0)),
            pl.BlockSpec(W_v.shape, lambda i: (0, 0)),
        ],
        out_specs=[
            pl.BlockSpec((2000, C), lambda i: (i, 0)),
            pl.BlockSpec((2000, C), lambda i: (i, 0)),
            pl.BlockSpec((2000, C), lambda i: (i, 0)),
            pl.BlockSpec((2000, NCLS), lambda i: (i, 0)),
        ],
        out_shape=[
            jax.ShapeDtypeStruct((n, C), jnp.float32),
            jax.ShapeDtypeStruct((n, C), jnp.float32),
            jax.ShapeDtypeStruct((n, C), jnp.float32),
            jax.ShapeDtypeStruct((n, NCLS), jnp.float32),
        ],
    )(node_feat, W_in, W_q, W_k, W_v)
    h = h + b_in
    Q = Q + b_q
    Kf = Kf + b_k
    V = V + b_v

    deg = jax.ops.segment_sum(jnp.ones((row.shape[0],), jnp.float32), col, num_segments=n)
    deg_inv = jnp.where(deg > 0, 1.0 / deg, 0.0)
    norm = deg_inv[row]

    M = Kf[:, :, None] * V[:, None, :]
    teleportM = jnp.sum(M, axis=0) / n
    teleportK = jnp.sum(Kf, axis=0, keepdims=True) / n
    teleportH = jnp.einsum('ni,ij->nj', Q, teleportM)
    teleportC = jnp.sum(Q * teleportK, axis=-1, keepdims=True) + CST
    teleportH = teleportH / teleportC
    hidden = V * hopwise[0]
    for hop in range(2):
        M = jax.ops.segment_sum(M[row] * norm[:, None, None], col, num_segments=n)
        Kf = jax.ops.segment_sum(Kf[row] * norm[:, None], col, num_segments=n)
        H = jnp.einsum('ni,nij->nj', Q, M)
        Cden = jnp.sum(Q * Kf, axis=-1, keepdims=True) + CST
        H = H / Cden
        hidden = hidden + hopwise[hop + 1] * H
    hidden = hidden + teleport * teleportH
    return hidden


# speedup vs baseline: 53.9654x; 53.9654x over previous
import functools

import jax
import jax.numpy as jnp
from jax import lax
from jax.experimental import pallas as pl
from jax.experimental.pallas import tpu as pltpu
from jax.experimental.pallas import tpu_sc as plsc

CST = 1e-05

N = 10000
E = 320000
C = 64
NCLS = 8
F = C * NCLS + C
NPH = 6
FP = 96
NPD = 10240
TR = NPD // 16
BLK = 2048
NBLK = NPD // BLK
NW = 32
EW = E // NW
CH = 200
NCH = EW // CH


def _one_plus_elu(x):
    return jnp.where(x > 0, 1.0 + x, jnp.exp(jnp.minimum(x, 0.0)))


def _tc1_body(x_ref, win_ref, bin_ref, wq_ref, bq_ref, wk_ref, bk_ref,
              wv_ref, bv_ref, q_ref, k_ref, v_ref, x0_ref, psum_ref):
    x = x_ref[...]
    h = jax.nn.relu(jnp.dot(x, win_ref[...], preferred_element_type=jnp.float32)
                    + bin_ref[...])
    q = _one_plus_elu(jnp.dot(h, wq_ref[...], preferred_element_type=jnp.float32)
                      + bq_ref[...])
    kf = _one_plus_elu(jnp.dot(h, wk_ref[...], preferred_element_type=jnp.float32)
                       + bk_ref[...])
    v = jnp.dot(h, wv_ref[...], preferred_element_type=jnp.float32) + bv_ref[...]
    q_ref[...] = q
    k_ref[...] = kf
    v_ref[...] = v
    for j in range(NCLS):
        x0_ref[:, j * C:(j + 1) * C] = kf * v[:, j:j + 1]
    x0_ref[:, NCLS * C:] = kf
    rows = pl.program_id(0) * BLK + lax.broadcasted_iota(jnp.int32, (BLK, 1), 0)
    mask = rows < N
    x0m = jnp.where(mask, x0_ref[...], 0.0)
    s = jnp.sum(x0m, axis=0, keepdims=True)
    psum_ref[...] = jnp.concatenate([s, jnp.zeros((7, F), jnp.float32)],
                                    axis=0)[None]


def _tc1(x_pad, W_in, b_in, W_q, b_q, W_k, b_k, W_v, b_v):
    full = lambda s: pl.BlockSpec(s, lambda i: (0, 0))
    return pl.pallas_call(
        _tc1_body,
        grid=(NBLK,),
        in_specs=[
            pl.BlockSpec((BLK, W_in.shape[0]), lambda i: (i, 0)),
            full(W_in.shape), full((1, C)),
            full(W_q.shape), full((1, C)),
            full(W_k.shape), full((1, C)),
            full(W_v.shape), full((1, NCLS)),
        ],
        out_specs=[
            pl.BlockSpec((BLK, C), lambda i: (i, 0)),
            pl.BlockSpec((BLK, C), lambda i: (i, 0)),
            pl.BlockSpec((BLK, NCLS), lambda i: (i, 0)),
            pl.BlockSpec((BLK, F), lambda i: (i, 0)),
            pl.BlockSpec((1, 8, F), lambda i: (i, 0, 0)),
        ],
        out_shape=[
            jax.ShapeDtypeStruct((NPD, C), jnp.float32),
            jax.ShapeDtypeStruct((NPD, C), jnp.float32),
            jax.ShapeDtypeStruct((NPD, NCLS), jnp.float32),
            jax.ShapeDtypeStruct((NPD, F), jnp.float32),
            jax.ShapeDtypeStruct((NBLK, 8, F), jnp.float32),
        ],
    )(x_pad, W_in, b_in, W_q, b_q, W_k, b_k, W_v, b_v)


def _deg_mesh():
    return plsc.VectorSubcoreMesh(core_axis_name="c", subcore_axis_name="s")


_SC_PARAMS = pltpu.CompilerParams(use_tc_tiling_on_sc=False)


def _sc_deg_body(edges_ref, z_ref, ones_ref, out_ref, deg_sh, colidx, ones,
                 sem0, sem1):
    cid = lax.axis_index("c")
    sid = lax.axis_index("s")
    wid = sid * 2 + cid
    pltpu.sync_copy(edges_ref.at[1, wid], colidx)
    pltpu.sync_copy(ones_ref, ones)
    pltpu.sync_copy(z_ref, deg_sh.at[pl.ds(sid * TR, TR)])
    plsc.subcore_barrier()
    descs = []
    for c in range(NCH):
        descs.append(pltpu.async_copy(
            ones, deg_sh.at[colidx.at[c]],
            sem0 if c % 2 == 0 else sem1, add=True))
        if c >= 1:
            descs[c - 1].wait()
    descs[NCH - 1].wait()
    plsc.subcore_barrier()
    pltpu.sync_copy(deg_sh.at[pl.ds(sid * TR, TR)],
                    out_ref.at[cid, pl.ds(sid * TR, TR)])


def _sc_deg(edges_r, z2, ones2):
    kfn = functools.partial(
        pl.kernel,
        out_type=jax.ShapeDtypeStruct((2, NPD, 16), jnp.float32),
        mesh=_deg_mesh(),
        scratch_types=[
            pltpu.VMEM_SHARED((NPD, 16), jnp.float32),
            pltpu.VMEM((NCH, CH), jnp.int32),
            pltpu.VMEM((CH, 16), jnp.float32),
            pltpu.SemaphoreType.DMA,
            pltpu.SemaphoreType.DMA,
        ],
        compiler_params=_SC_PARAMS,
    )(_sc_deg_body)
    return kfn(edges_r, z2, ones2)


def _split_tables(xn, trefs):
    for p in range(NPH):
        trefs[p][...] = xn[:, p * FP:(p + 1) * FP]


def _tc2_body(degp_ref, x0_ref, dinv_ref, t0_ref, t1_ref, t2_ref, t3_ref,
              t4_ref, t5_ref):
    deg = degp_ref[0, 0, :, 0:1] + degp_ref[1, 0, :, 0:1]
    dinv = jnp.where(deg > 0, 1.0 / jnp.maximum(deg, 1e-30), 0.0)
    dinv_ref[...] = dinv
    xn = x0_ref[...] * dinv
    _split_tables(xn, [t0_ref, t1_ref, t2_ref, t3_ref, t4_ref, t5_ref])


def _tc2(degp_r, x0):
    outs = pl.pallas_call(
        _tc2_body,
        grid=(NBLK,),
        in_specs=[
            pl.BlockSpec((2, 1, BLK, 16), lambda i: (0, i, 0, 0)),
            pl.BlockSpec((BLK, F), lambda i: (i, 0)),
        ],
        out_specs=[pl.BlockSpec((BLK, 1), lambda i: (i, 0))] +
                  [pl.BlockSpec((BLK, FP), lambda i: (i, 0))] * NPH,
        out_shape=[jax.ShapeDtypeStruct((NPD, 1), jnp.float32)] +
                  [jax.ShapeDtypeStruct((NPD, FP), jnp.float32)] * NPH,
    )(degp_r, x0)
    return outs[0], outs[1:]


def _sc_prop_body(edges_ref, t0, t1, t2, t3, t4, t5, z2_ref, out_ref,
                  xa_sh, rowidx, colidx, buf0, buf1,
                  gs0, gs1, ss0, ss1):
    cid = lax.axis_index("c")
    sid = lax.axis_index("s")
    wid = sid * 2 + cid
    pltpu.sync_copy(edges_ref.at[0, wid], rowidx)
    pltpu.sync_copy(edges_ref.at[1, wid], colidx)
    tables = (t0, t1, t2, t3, t4, t5)
    bufs = (buf0, buf1)
    gsems = (gs0, gs1)
    ssems = (ss0, ss1)
    for p in range(NPH):
        tp = tables[p]
        pltpu.sync_copy(z2_ref, xa_sh.at[pl.ds(sid * TR, TR)])
        plsc.subcore_barrier()
        sd = [None, None]
        for c in range(NCH):
            b = c % 2
            if sd[b] is not None:
                sd[b].wait()
            gd = pltpu.async_copy(tp.at[rowidx.at[c]], bufs[b], gsems[b])
            gd.wait()
            sd[b] = pltpu.async_copy(bufs[b], xa_sh.at[colidx.at[c]],
                                     ssems[b], add=True)
        for b in range(2):
            if sd[b] is not None:
                sd[b].wait()
        plsc.subcore_barrier()
        pltpu.sync_copy(xa_sh.at[pl.ds(sid * TR, TR)],
                        out_ref.at[cid, p, pl.ds(sid * TR, TR)])
        plsc.subcore_barrier()


def _sc_prop(edges_r, tables, z2):
    kfn = functools.partial(
        pl.kernel,
        out_type=jax.ShapeDtypeStruct((2, NPH, NPD, FP), jnp.float32),
        mesh=_deg_mesh(),
        scratch_types=[
            pltpu.VMEM_SHARED((NPD, FP), jnp.float32),
            pltpu.VMEM((NCH, CH), jnp.int32),
            pltpu.VMEM((NCH, CH), jnp.int32),
            pltpu.VMEM((CH, FP), jnp.float32),
            pltpu.VMEM((CH, FP), jnp.float32),
            pltpu.SemaphoreType.DMA,
            pltpu.SemaphoreType.DMA,
            pltpu.SemaphoreType.DMA,
            pltpu.SemaphoreType.DMA,
        ],
        compiler_params=_SC_PARAMS,
    )(_sc_prop_body)
    return kfn(edges_r, *tables, z2)


def _tc3_body(outp_ref, q_ref, dinv_ref, h_ref, cden_ref,
              t0_ref, t1_ref, t2_ref, t3_ref, t4_ref, t5_ref):
    x1 = jnp.concatenate(
        [outp_ref[0, p] + outp_ref[1, p] for p in range(NPH)], axis=-1)
    q = q_ref[...]
    cols = [jnp.sum(q * x1[:, j * C:(j + 1) * C], axis=1, keepdims=True)
            for j in range(NCLS)]
    h_ref[...] = jnp.concatenate(cols, axis=-1)
    cden_ref[...] = jnp.sum(q * x1[:, NCLS * C:], axis=1, keepdims=True) + CST
    xn = x1 * dinv_ref[...]
    _split_tables(xn, [t0_ref, t1_ref, t2_ref, t3_ref, t4_ref, t5_ref])


def _tc3(outp, q, dinv):
    outs = pl.pallas_call(
        _tc3_body,
        grid=(NBLK,),
        in_specs=[
            pl.BlockSpec((2, NPH, BLK, FP), lambda i: (0, 0, i, 0)),
            pl.BlockSpec((BLK, C), lambda i: (i, 0)),
            pl.BlockSpec((BLK, 1), lambda i: (i, 0)),
        ],
        out_specs=[pl.BlockSpec((BLK, NCLS), lambda i: (i, 0)),
                   pl.BlockSpec((BLK, 1), lambda i: (i, 0))] +
                  [pl.BlockSpec((BLK, FP), lambda i: (i, 0))] * NPH,
        out_shape=[jax.ShapeDtypeStruct((NPD, NCLS), jnp.float32),
                   jax.ShapeDtypeStruct((NPD, 1), jnp.float32)] +
                  [jax.ShapeDtypeStruct((NPD, FP), jnp.float32)] * NPH,
    )(outp, q, dinv)
    return outs[0], outs[1], outs[2:2 + NPH]


def _tc4_body(outp_ref, q_ref, v_ref, h1_ref, cd1_ref, psum_ref,
              hop_ref, tel_ref, hid_ref):
    x2 = jnp.concatenate(
        [outp_ref[0, p] + outp_ref[1, p] for p in range(NPH)], axis=-1)
    q = q_ref[...]
    cols = [jnp.sum(q * x2[:, j * C:(j + 1) * C], axis=1, keepdims=True)
            for j in range(NCLS)]
    h2 = jnp.concatenate(cols, axis=-1)
    cd2 = jnp.sum(q * x2[:, NCLS * C:], axis=1, keepdims=True) + CST
    tvec = jnp.sum(psum_ref[...], axis=(0, 1)) / float(N)
    tcols = [jnp.dot(q, tvec[j * C:(j + 1) * C][:, None],
                     preferred_element_type=jnp.float32)
             for j in range(NCLS)]
    th = jnp.concatenate(tcols, axis=-1)
    tc = jnp.dot(q, tvec[NCLS * C:][:, None],
                 preferred_element_type=jnp.float32) + CST
    hid_ref[...] = (hop_ref[0] * v_ref[...]
                    + hop_ref[1] * (h1_ref[...] / cd1_ref[...])
                    + hop_ref[2] * (h2 / cd2)
                    + tel_ref[0] * (th / tc))


def _tc4(outp2, q, v, h1, cd1, psum, hopwise, teleport):
    return pl.pallas_call(
        _tc4_body,
        grid=(NBLK,),
        in_specs=[
            pl.BlockSpec((2, NPH, BLK, FP), lambda i: (0, 0, i, 0)),
            pl.BlockSpec((BLK, C), lambda i: (i, 0)),
            pl.BlockSpec((BLK, NCLS), lambda i: (i, 0)),
            pl.BlockSpec((BLK, NCLS), lambda i: (i, 0)),
            pl.BlockSpec((BLK, 1), lambda i: (i, 0)),
            pl.BlockSpec((NBLK, 8, F), lambda i: (0, 0, 0)),
            pl.BlockSpec(memory_space=pltpu.SMEM),
            pl.BlockSpec(memory_space=pltpu.SMEM),
        ],
        out_specs=pl.BlockSpec((BLK, NCLS), lambda i: (i, 0)),
        out_shape=jax.ShapeDtypeStruct((NPD, NCLS), jnp.float32),
    )(outp2, q, v, h1, cd1, psum, hopwise, teleport)


def kernel(node_feat, edge_index, W_in, b_in, W_q, b_q, W_k, b_k, W_v, b_v,
           hopwise, teleport):
    assert node_feat.shape[0] == N and edge_index.shape == (2, E)
    x_pad = jnp.pad(node_feat, ((0, NPD - N), (0, 0)))
    edges_r = edge_index.reshape(2, NW, NCH, CH)

    z2 = jnp.zeros((TR, FP), jnp.float32)
    z16 = jnp.zeros((TR, 16), jnp.float32)
    ones2 = jnp.ones((CH, 16), jnp.float32)

    q, kf, v, x0, psum = _tc1(
        x_pad, W_in, b_in.reshape(1, C), W_q, b_q.reshape(1, C),
        W_k, b_k.reshape(1, C), W_v, b_v.reshape(1, NCLS))

    degp = _sc_deg(edges_r, z16, ones2)
    degp_r = degp.reshape(2, NBLK, BLK, 16)

    dinv, tables0 = _tc2(degp_r, x0)

    outp1 = _sc_prop(edges_r, tables0, z2)
    h1, cd1, tables1 = _tc3(outp1, q, dinv)
    outp2 = _sc_prop(edges_r, tables1, z2)

    hid = _tc4(outp2, q, v, h1, cd1, psum, hopwise, teleport)
    return hid[:N]


# speedup vs baseline: 59.8176x; 1.1084x over previous
import functools

import jax
import jax.numpy as jnp
from jax import lax
from jax.experimental import pallas as pl
from jax.experimental.pallas import tpu as pltpu
from jax.experimental.pallas import tpu_sc as plsc

CST = 1e-05

N = 10000
E = 320000
C = 64
NCLS = 8
F = C * NCLS + C
NPH = 6
FP = 96
NPD = 10240
TR = NPD // 16
BLK = 2048
NBLK = NPD // BLK
NW = 32
EW = E // NW
CH = 200
NCH = EW // CH

_SC_PARAMS = pltpu.CompilerParams(use_tc_tiling_on_sc=False)


def _one_plus_elu(x):
    return jnp.where(x > 0, 1.0 + x, jnp.exp(jnp.minimum(x, 0.0)))


def _mesh():
    return plsc.VectorSubcoreMesh(core_axis_name="c", subcore_axis_name="s")


def _sc_deg_body(edges_ref, z_ref, ones_ref, out_ref, deg_sh, colidx, ones,
                 sem0, sem1):
    cid = lax.axis_index("c")
    sid = lax.axis_index("s")
    wid = sid * 2 + cid
    pltpu.sync_copy(edges_ref.at[1, wid], colidx)
    pltpu.sync_copy(ones_ref, ones)
    pltpu.sync_copy(z_ref, deg_sh.at[pl.ds(sid * TR, TR)])
    plsc.subcore_barrier()
    descs = []
    for c in range(NCH):
        descs.append(pltpu.async_copy(
            ones, deg_sh.at[colidx.at[c]],
            sem0 if c % 2 == 0 else sem1, add=True))
        if c >= 1:
            descs[c - 1].wait()
    descs[NCH - 1].wait()
    plsc.subcore_barrier()
    pltpu.sync_copy(deg_sh.at[pl.ds(sid * TR, TR)],
                    out_ref.at[cid, pl.ds(sid * TR, TR)])


def _sc_deg(edges_r, z16, ones2):
    kfn = functools.partial(
        pl.kernel,
        out_type=jax.ShapeDtypeStruct((2, NPD, 16), jnp.float32),
        mesh=_mesh(),
        scratch_types=[
            pltpu.VMEM_SHARED((NPD, 16), jnp.float32),
            pltpu.VMEM((NCH, CH), jnp.int32),
            pltpu.VMEM((CH, 16), jnp.float32),
            pltpu.SemaphoreType.DMA,
            pltpu.SemaphoreType.DMA,
        ],
        compiler_params=_SC_PARAMS,
    )(_sc_deg_body)
    return kfn(edges_r, z16, ones2)


def _split_tables(xn, trefs):
    for p in range(NPH):
        trefs[p][...] = xn[:, p * FP:(p + 1) * FP]


def _tc1_body(x_ref, win_ref, bin_ref, wq_ref, bq_ref, wk_ref, bk_ref,
              wv_ref, bv_ref, degp_ref,
              q_ref, v_ref, dinv_ref, psum_ref,
              t0_ref, t1_ref, t2_ref, t3_ref, t4_ref, t5_ref):
    x = x_ref[...]
    h = jax.nn.relu(jnp.dot(x, win_ref[...], preferred_element_type=jnp.float32)
                    + bin_ref[...])
    q = _one_plus_elu(jnp.dot(h, wq_ref[...], preferred_element_type=jnp.float32)
                      + bq_ref[...])
    kf = _one_plus_elu(jnp.dot(h, wk_ref[...], preferred_element_type=jnp.float32)
                       + bk_ref[...])
    v = jnp.dot(h, wv_ref[...], preferred_element_type=jnp.float32) + bv_ref[...]
    q_ref[...] = q
    v_ref[...] = v
    x0 = jnp.concatenate([kf * v[:, j:j + 1] for j in range(NCLS)] + [kf],
                         axis=-1)
    rows = pl.program_id(0) * BLK + lax.broadcasted_iota(jnp.int32, (BLK, 1), 0)
    x0m = jnp.where(rows < N, x0, 0.0)
    s = jnp.sum(x0m, axis=0, keepdims=True)
    psum_ref[...] = jnp.concatenate([s, jnp.zeros((7, F), jnp.float32)],
                                    axis=0)[None]
    deg = degp_ref[0, 0, :, 0:1] + degp_ref[1, 0, :, 0:1]
    dinv = jnp.where(deg > 0, 1.0 / jnp.maximum(deg, 1e-30), 0.0)
    dinv_ref[...] = dinv
    _split_tables(x0 * dinv, [t0_ref, t1_ref, t2_ref, t3_ref, t4_ref, t5_ref])


def _tc1(x_pad, W_in, b_in, W_q, b_q, W_k, b_k, W_v, b_v, degp_r):
    full = lambda s: pl.BlockSpec(s, lambda i: (0, 0))
    return pl.pallas_call(
        _tc1_body,
        grid=(NBLK,),
        in_specs=[
            pl.BlockSpec((BLK, W_in.shape[0]), lambda i: (i, 0)),
            full(W_in.shape), full((1, C)),
            full(W_q.shape), full((1, C)),
            full(W_k.shape), full((1, C)),
            full(W_v.shape), full((1, NCLS)),
            pl.BlockSpec((2, 1, BLK, 16), lambda i: (0, i, 0, 0)),
        ],
        out_specs=[
            pl.BlockSpec((BLK, C), lambda i: (i, 0)),
            pl.BlockSpec((BLK, NCLS), lambda i: (i, 0)),
            pl.BlockSpec((BLK, 1), lambda i: (i, 0)),
            pl.BlockSpec((1, 8, F), lambda i: (i, 0, 0)),
        ] + [pl.BlockSpec((BLK, FP), lambda i: (i, 0))] * NPH,
        out_shape=[
            jax.ShapeDtypeStruct((NPD, C), jnp.float32),
            jax.ShapeDtypeStruct((NPD, NCLS), jnp.float32),
            jax.ShapeDtypeStruct((NPD, 1), jnp.float32),
            jax.ShapeDtypeStruct((NBLK, 8, F), jnp.float32),
        ] + [jax.ShapeDtypeStruct((NPD, FP), jnp.float32)] * NPH,
    )(x_pad, W_in, b_in, W_q, b_q, W_k, b_k, W_v, b_v, degp_r)


def _sc_prop_body(edges_ref, t0, t1, t2, t3, t4, t5, z2_ref, out_ref,
                  xa_sh, rowidx, colidx, buf0, buf1,
                  gs0, gs1, ss0, ss1):
    cid = lax.axis_index("c")
    sid = lax.axis_index("s")
    wid = sid * 2 + cid
    pltpu.sync_copy(edges_ref.at[0, wid], rowidx)
    pltpu.sync_copy(edges_ref.at[1, wid], colidx)
    tables = (t0, t1, t2, t3, t4, t5)
    bufs = (buf0, buf1)
    gsems = (gs0, gs1)
    ssems = (ss0, ss1)
    for p in range(NPH):
        tp = tables[p]
        pltpu.sync_copy(z2_ref, xa_sh.at[pl.ds(sid * TR, TR)])
        plsc.subcore_barrier()
        gd = [None, None]
        sd = [None, None]
        gd[0] = pltpu.async_copy(tp.at[rowidx.at[0]], bufs[0], gsems[0])
        for c in range(NCH):
            b = c % 2
            nb = (c + 1) % 2
            if sd[nb] is not None:
                sd[nb].wait()
                sd[nb] = None
            if c + 1 < NCH:
                gd[nb] = pltpu.async_copy(tp.at[rowidx.at[c + 1]],
                                          bufs[nb], gsems[nb])
            gd[b].wait()
            sd[b] = pltpu.async_copy(bufs[b], xa_sh.at[colidx.at[c]],
                                     ssems[b], add=True)
        for b in range(2):
            if sd[b] is not None:
                sd[b].wait()
        plsc.subcore_barrier()
        pltpu.sync_copy(xa_sh.at[pl.ds(sid * TR, TR)],
                        out_ref.at[cid, p, pl.ds(sid * TR, TR)])
        plsc.subcore_barrier()


def _sc_prop(edges_r, tables, z2):
    kfn = functools.partial(
        pl.kernel,
        out_type=jax.ShapeDtypeStruct((2, NPH, NPD, FP), jnp.float32),
        mesh=_mesh(),
        scratch_types=[
            pltpu.VMEM_SHARED((NPD, FP), jnp.float32),
            pltpu.VMEM((NCH, CH), jnp.int32),
            pltpu.VMEM((NCH, CH), jnp.int32),
            pltpu.VMEM((CH, FP), jnp.float32),
            pltpu.VMEM((CH, FP), jnp.float32),
            pltpu.SemaphoreType.DMA,
            pltpu.SemaphoreType.DMA,
            pltpu.SemaphoreType.DMA,
            pltpu.SemaphoreType.DMA,
        ],
        compiler_params=_SC_PARAMS,
    )(_sc_prop_body)
    return kfn(edges_r, *tables, z2)


def _tc2_body(outp_ref, q_ref, dinv_ref, h_ref, cden_ref,
              t0_ref, t1_ref, t2_ref, t3_ref, t4_ref, t5_ref):
    x1 = jnp.concatenate(
        [outp_ref[0, p] + outp_ref[1, p] for p in range(NPH)], axis=-1)
    q = q_ref[...]
    cols = [jnp.sum(q * x1[:, j * C:(j + 1) * C], axis=1, keepdims=True)
            for j in range(NCLS)]
    h_ref[...] = jnp.concatenate(cols, axis=-1)
    cden_ref[...] = jnp.sum(q * x1[:, NCLS * C:], axis=1, keepdims=True) + CST
    _split_tables(x1 * dinv_ref[...],
                  [t0_ref, t1_ref, t2_ref, t3_ref, t4_ref, t5_ref])


def _tc2(outp, q, dinv):
    outs = pl.pallas_call(
        _tc2_body,
        grid=(NBLK,),
        in_specs=[
            pl.BlockSpec((2, NPH, BLK, FP), lambda i: (0, 0, i, 0)),
            pl.BlockSpec((BLK, C), lambda i: (i, 0)),
            pl.BlockSpec((BLK, 1), lambda i: (i, 0)),
        ],
        out_specs=[pl.BlockSpec((BLK, NCLS), lambda i: (i, 0)),
                   pl.BlockSpec((BLK, 1), lambda i: (i, 0))] +
                  [pl.BlockSpec((BLK, FP), lambda i: (i, 0))] * NPH,
        out_shape=[jax.ShapeDtypeStruct((NPD, NCLS), jnp.float32),
                   jax.ShapeDtypeStruct((NPD, 1), jnp.float32)] +
                  [jax.ShapeDtypeStruct((NPD, FP), jnp.float32)] * NPH,
    )(outp, q, dinv)
    return outs[0], outs[1], outs[2:2 + NPH]


def _tc3_body(outp_ref, q_ref, v_ref, h1_ref, cd1_ref, psum_ref,
              hop_ref, tel_ref, hid_ref):
    x2 = jnp.concatenate(
        [outp_ref[0, p] + outp_ref[1, p] for p in range(NPH)], axis=-1)
    q = q_ref[...]
    cols = [jnp.sum(q * x2[:, j * C:(j + 1) * C], axis=1, keepdims=True)
            for j in range(NCLS)]
    h2 = jnp.concatenate(cols, axis=-1)
    cd2 = jnp.sum(q * x2[:, NCLS * C:], axis=1, keepdims=True) + CST
    tvec = jnp.sum(psum_ref[...], axis=(0, 1)) / float(N)
    tcols = [jnp.dot(q, tvec[j * C:(j + 1) * C][:, None],
                     preferred_element_type=jnp.float32)
             for j in range(NCLS)]
    th = jnp.concatenate(tcols, axis=-1)
    tc = jnp.dot(q, tvec[NCLS * C:][:, None],
                 preferred_element_type=jnp.float32) + CST
    hid_ref[...] = (hop_ref[0] * v_ref[...]
                    + hop_ref[1] * (h1_ref[...] / cd1_ref[...])
                    + hop_ref[2] * (h2 / cd2)
                    + tel_ref[0] * (th / tc))


def _tc3(outp2, q, v, h1, cd1, psum, hopwise, teleport):
    return pl.pallas_call(
        _tc3_body,
        grid=(NBLK,),
        in_specs=[
            pl.BlockSpec((2, NPH, BLK, FP), lambda i: (0, 0, i, 0)),
            pl.BlockSpec((BLK, C), lambda i: (i, 0)),
            pl.BlockSpec((BLK, NCLS), lambda i: (i, 0)),
            pl.BlockSpec((BLK, NCLS), lambda i: (i, 0)),
            pl.BlockSpec((BLK, 1), lambda i: (i, 0)),
            pl.BlockSpec((NBLK, 8, F), lambda i: (0, 0, 0)),
            pl.BlockSpec(memory_space=pltpu.SMEM),
            pl.BlockSpec(memory_space=pltpu.SMEM),
        ],
        out_specs=pl.BlockSpec((BLK, NCLS), lambda i: (i, 0)),
        out_shape=jax.ShapeDtypeStruct((NPD, NCLS), jnp.float32),
    )(outp2, q, v, h1, cd1, psum, hopwise, teleport)


def kernel(node_feat, edge_index, W_in, b_in, W_q, b_q, W_k, b_k, W_v, b_v,
           hopwise, teleport):
    assert node_feat.shape[0] == N and edge_index.shape == (2, E)
    x_pad = jnp.pad(node_feat, ((0, NPD - N), (0, 0)))
    edges_r = edge_index.reshape(2, NW, NCH, CH)

    z2 = jnp.zeros((TR, FP), jnp.float32)
    z16 = jnp.zeros((TR, 16), jnp.float32)
    ones2 = jnp.ones((CH, 16), jnp.float32)

    degp = _sc_deg(edges_r, z16, ones2)
    degp_r = degp.reshape(2, NBLK, BLK, 16)

    outs = _tc1(x_pad, W_in, b_in.reshape(1, C), W_q, b_q.reshape(1, C),
                W_k, b_k.reshape(1, C), W_v, b_v.reshape(1, NCLS), degp_r)
    q, v, dinv, psum = outs[:4]
    tables0 = outs[4:]

    outp1 = _sc_prop(edges_r, tables0, z2)
    h1, cd1, tables1 = _tc2(outp1, q, dinv)
    outp2 = _sc_prop(edges_r, tables1, z2)

    hid = _tc3(outp2, q, v, h1, cd1, psum, hopwise, teleport)
    return hid[:N]


# speedup vs baseline: 71.1704x; 1.1898x over previous
import functools

import jax
import jax.numpy as jnp
from jax import lax
from jax.experimental import pallas as pl
from jax.experimental.pallas import tpu as pltpu
from jax.experimental.pallas import tpu_sc as plsc

CST = 1e-05

N = 10000
E = 320000
C = 64
NCLS = 8
F = C * NCLS + C
NPH = 6
FP = 96
NPD = 10240
TR = NPD // 16
BLK = 2048
NBLK = NPD // BLK
NW = 32
EW = E // NW
CH = 200
NCH = EW // CH

_SC_PARAMS = pltpu.CompilerParams(use_tc_tiling_on_sc=False)
DT = jnp.bfloat16


def _one_plus_elu(x):
    return jnp.where(x > 0, 1.0 + x, jnp.exp(jnp.minimum(x, 0.0)))


def _mesh():
    return plsc.VectorSubcoreMesh(core_axis_name="c", subcore_axis_name="s")


def _sc_deg_body(edges_ref, z_ref, ones_ref, out_ref, deg_sh, colidx, ones,
                 sem0, sem1):
    cid = lax.axis_index("c")
    sid = lax.axis_index("s")
    wid = sid * 2 + cid
    pltpu.sync_copy(edges_ref.at[1, wid], colidx)
    pltpu.sync_copy(ones_ref, ones)
    pltpu.sync_copy(z_ref, deg_sh.at[pl.ds(sid * TR, TR)])
    plsc.subcore_barrier()
    descs = []
    for c in range(NCH):
        descs.append(pltpu.async_copy(
            ones, deg_sh.at[colidx.at[c]],
            sem0 if c % 2 == 0 else sem1, add=True))
        if c >= 1:
            descs[c - 1].wait()
    descs[NCH - 1].wait()
    plsc.subcore_barrier()
    pltpu.sync_copy(deg_sh.at[pl.ds(sid * TR, TR)],
                    out_ref.at[cid, pl.ds(sid * TR, TR)])


def _sc_deg(edges_r, z16, ones2):
    kfn = functools.partial(
        pl.kernel,
        out_type=jax.ShapeDtypeStruct((2, NPD, 16), jnp.float32),
        mesh=_mesh(),
        scratch_types=[
            pltpu.VMEM_SHARED((NPD, 16), jnp.float32),
            pltpu.VMEM((NCH, CH), jnp.int32),
            pltpu.VMEM((CH, 16), jnp.float32),
            pltpu.SemaphoreType.DMA,
            pltpu.SemaphoreType.DMA,
        ],
        compiler_params=_SC_PARAMS,
    )(_sc_deg_body)
    return kfn(edges_r, z16, ones2)


def _split_tables(xn, trefs):
    xn = xn.astype(DT)
    for p in range(NPH):
        trefs[p][...] = xn[:, p * FP:(p + 1) * FP]


def _tc1_body(x_ref, win_ref, bin_ref, wq_ref, bq_ref, wk_ref, bk_ref,
              wv_ref, bv_ref, degp_ref,
              q_ref, v_ref, dinv_ref, psum_ref,
              t0_ref, t1_ref, t2_ref, t3_ref, t4_ref, t5_ref):
    x = x_ref[...]
    h = jax.nn.relu(jnp.dot(x, win_ref[...], preferred_element_type=jnp.float32)
                    + bin_ref[...])
    q = _one_plus_elu(jnp.dot(h, wq_ref[...], preferred_element_type=jnp.float32)
                      + bq_ref[...])
    kf = _one_plus_elu(jnp.dot(h, wk_ref[...], preferred_element_type=jnp.float32)
                       + bk_ref[...])
    v = jnp.dot(h, wv_ref[...], preferred_element_type=jnp.float32) + bv_ref[...]
    q_ref[...] = q
    v_ref[...] = v
    x0 = jnp.concatenate([kf * v[:, j:j + 1] for j in range(NCLS)] + [kf],
                         axis=-1)
    rows = pl.program_id(0) * BLK + lax.broadcasted_iota(jnp.int32, (BLK, 1), 0)
    x0m = jnp.where(rows < N, x0, 0.0)
    s = jnp.sum(x0m, axis=0, keepdims=True)
    psum_ref[...] = jnp.concatenate([s, jnp.zeros((7, F), jnp.float32)],
                                    axis=0)[None]
    deg = degp_ref[0, 0, :, 0:1] + degp_ref[1, 0, :, 0:1]
    dinv = jnp.where(deg > 0, 1.0 / jnp.maximum(deg, 1e-30), 0.0)
    dinv_ref[...] = dinv
    _split_tables(x0 * dinv, [t0_ref, t1_ref, t2_ref, t3_ref, t4_ref, t5_ref])


def _tc1(x_pad, W_in, b_in, W_q, b_q, W_k, b_k, W_v, b_v, degp_r):
    full = lambda s: pl.BlockSpec(s, lambda i: (0, 0))
    return pl.pallas_call(
        _tc1_body,
        grid=(NBLK,),
        in_specs=[
            pl.BlockSpec((BLK, W_in.shape[0]), lambda i: (i, 0)),
            full(W_in.shape), full((1, C)),
            full(W_q.shape), full((1, C)),
            full(W_k.shape), full((1, C)),
            full(W_v.shape), full((1, NCLS)),
            pl.BlockSpec((2, 1, BLK, 16), lambda i: (0, i, 0, 0)),
        ],
        out_specs=[
            pl.BlockSpec((BLK, C), lambda i: (i, 0)),
            pl.BlockSpec((BLK, NCLS), lambda i: (i, 0)),
            pl.BlockSpec((BLK, 1), lambda i: (i, 0)),
            pl.BlockSpec((1, 8, F), lambda i: (i, 0, 0)),
        ] + [pl.BlockSpec((BLK, FP), lambda i: (i, 0))] * NPH,
        out_shape=[
            jax.ShapeDtypeStruct((NPD, C), jnp.float32),
            jax.ShapeDtypeStruct((NPD, NCLS), jnp.float32),
            jax.ShapeDtypeStruct((NPD, 1), jnp.float32),
            jax.ShapeDtypeStruct((NBLK, 8, F), jnp.float32),
        ] + [jax.ShapeDtypeStruct((NPD, FP), DT)] * NPH,
    )(x_pad, W_in, b_in, W_q, b_q, W_k, b_k, W_v, b_v, degp_r)


def _sc_prop_body(edges_ref, t0, t1, t2, t3, t4, t5, z2_ref, out_ref,
                  xa_sh, rowidx, colidx, buf0, buf1,
                  gs0, gs1, ss0, ss1):
    cid = lax.axis_index("c")
    sid = lax.axis_index("s")
    wid = sid * 2 + cid
    pltpu.sync_copy(edges_ref.at[0, wid], rowidx)
    pltpu.sync_copy(edges_ref.at[1, wid], colidx)
    tables = (t0, t1, t2, t3, t4, t5)
    bufs = (buf0, buf1)
    gsems = (gs0, gs1)
    ssems = (ss0, ss1)
    for p in range(NPH):
        tp = tables[p]
        pltpu.sync_copy(z2_ref, xa_sh.at[pl.ds(sid * TR, TR)])
        plsc.subcore_barrier()
        gd = [None, None]
        sd = [None, None]
        gd[0] = pltpu.async_copy(tp.at[rowidx.at[0]], bufs[0], gsems[0])
        for c in range(NCH):
            b = c % 2
            nb = (c + 1) % 2
            if sd[nb] is not None:
                sd[nb].wait()
                sd[nb] = None
            if c + 1 < NCH:
                gd[nb] = pltpu.async_copy(tp.at[rowidx.at[c + 1]],
                                          bufs[nb], gsems[nb])
            gd[b].wait()
            sd[b] = pltpu.async_copy(bufs[b], xa_sh.at[colidx.at[c]],
                                     ssems[b], add=True)
        for b in range(2):
            if sd[b] is not None:
                sd[b].wait()
        plsc.subcore_barrier()
        pltpu.sync_copy(xa_sh.at[pl.ds(sid * TR, TR)],
                        out_ref.at[cid, p, pl.ds(sid * TR, TR)])
        plsc.subcore_barrier()


def _sc_prop(edges_r, tables, z2):
    kfn = functools.partial(
        pl.kernel,
        out_type=jax.ShapeDtypeStruct((2, NPH, NPD, FP), DT),
        mesh=_mesh(),
        scratch_types=[
            pltpu.VMEM_SHARED((NPD, FP), DT),
            pltpu.VMEM((NCH, CH), jnp.int32),
            pltpu.VMEM((NCH, CH), jnp.int32),
            pltpu.VMEM((CH, FP), DT),
            pltpu.VMEM((CH, FP), DT),
            pltpu.SemaphoreType.DMA,
            pltpu.SemaphoreType.DMA,
            pltpu.SemaphoreType.DMA,
            pltpu.SemaphoreType.DMA,
        ],
        compiler_params=_SC_PARAMS,
    )(_sc_prop_body)
    return kfn(edges_r, *tables, z2)


def _tc2_body(outp_ref, q_ref, dinv_ref, h_ref, cden_ref,
              t0_ref, t1_ref, t2_ref, t3_ref, t4_ref, t5_ref):
    x1 = jnp.concatenate(
        [outp_ref[0, p].astype(jnp.float32) +
         outp_ref[1, p].astype(jnp.float32) for p in range(NPH)], axis=-1)
    q = q_ref[...]
    cols = [jnp.sum(q * x1[:, j * C:(j + 1) * C], axis=1, keepdims=True)
            for j in range(NCLS)]
    h_ref[...] = jnp.concatenate(cols, axis=-1)
    cden_ref[...] = jnp.sum(q * x1[:, NCLS * C:], axis=1, keepdims=True) + CST
    _split_tables(x1 * dinv_ref[...],
                  [t0_ref, t1_ref, t2_ref, t3_ref, t4_ref, t5_ref])


def _tc2(outp, q, dinv):
    outs = pl.pallas_call(
        _tc2_body,
        grid=(NBLK,),
        in_specs=[
            pl.BlockSpec((2, NPH, BLK, FP), lambda i: (0, 0, i, 0)),
            pl.BlockSpec((BLK, C), lambda i: (i, 0)),
            pl.BlockSpec((BLK, 1), lambda i: (i, 0)),
        ],
        out_specs=[pl.BlockSpec((BLK, NCLS), lambda i: (i, 0)),
                   pl.BlockSpec((BLK, 1), lambda i: (i, 0))] +
                  [pl.BlockSpec((BLK, FP), lambda i: (i, 0))] * NPH,
        out_shape=[jax.ShapeDtypeStruct((NPD, NCLS), jnp.float32),
                   jax.ShapeDtypeStruct((NPD, 1), jnp.float32)] +
                  [jax.ShapeDtypeStruct((NPD, FP), DT)] * NPH,
    )(outp, q, dinv)
    return outs[0], outs[1], outs[2:2 + NPH]


def _tc3_body(outp_ref, q_ref, v_ref, h1_ref, cd1_ref, psum_ref,
              hop_ref, tel_ref, hid_ref):
    x2 = jnp.concatenate(
        [outp_ref[0, p].astype(jnp.float32) +
         outp_ref[1, p].astype(jnp.float32) for p in range(NPH)], axis=-1)
    q = q_ref[...]
    cols = [jnp.sum(q * x2[:, j * C:(j + 1) * C], axis=1, keepdims=True)
            for j in range(NCLS)]
    h2 = jnp.concatenate(cols, axis=-1)
    cd2 = jnp.sum(q * x2[:, NCLS * C:], axis=1, keepdims=True) + CST
    tvec = jnp.sum(psum_ref[...], axis=(0, 1)) / float(N)
    tcols = [jnp.dot(q, tvec[j * C:(j + 1) * C][:, None],
                     preferred_element_type=jnp.float32)
             for j in range(NCLS)]
    th = jnp.concatenate(tcols, axis=-1)
    tc = jnp.dot(q, tvec[NCLS * C:][:, None],
                 preferred_element_type=jnp.float32) + CST
    hid_ref[...] = (hop_ref[0] * v_ref[...]
                    + hop_ref[1] * (h1_ref[...] / cd1_ref[...])
                    + hop_ref[2] * (h2 / cd2)
                    + tel_ref[0] * (th / tc))


def _tc3(outp2, q, v, h1, cd1, psum, hopwise, teleport):
    return pl.pallas_call(
        _tc3_body,
        grid=(NBLK,),
        in_specs=[
            pl.BlockSpec((2, NPH, BLK, FP), lambda i: (0, 0, i, 0)),
            pl.BlockSpec((BLK, C), lambda i: (i, 0)),
            pl.BlockSpec((BLK, NCLS), lambda i: (i, 0)),
            pl.BlockSpec((BLK, NCLS), lambda i: (i, 0)),
            pl.BlockSpec((BLK, 1), lambda i: (i, 0)),
            pl.BlockSpec((NBLK, 8, F), lambda i: (0, 0, 0)),
            pl.BlockSpec(memory_space=pltpu.SMEM),
            pl.BlockSpec(memory_space=pltpu.SMEM),
        ],
        out_specs=pl.BlockSpec((BLK, NCLS), lambda i: (i, 0)),
        out_shape=jax.ShapeDtypeStruct((NPD, NCLS), jnp.float32),
    )(outp2, q, v, h1, cd1, psum, hopwise, teleport)


def kernel(node_feat, edge_index, W_in, b_in, W_q, b_q, W_k, b_k, W_v, b_v,
           hopwise, teleport):
    assert node_feat.shape[0] == N and edge_index.shape == (2, E)
    x_pad = jnp.pad(node_feat, ((0, NPD - N), (0, 0)))
    edges_r = edge_index.reshape(2, NW, NCH, CH)

    z2 = jnp.zeros((TR, FP), DT)
    z16 = jnp.zeros((TR, 16), jnp.float32)
    ones2 = jnp.ones((CH, 16), jnp.float32)

    degp = _sc_deg(edges_r, z16, ones2)
    degp_r = degp.reshape(2, NBLK, BLK, 16)

    outs = _tc1(x_pad, W_in, b_in.reshape(1, C), W_q, b_q.reshape(1, C),
                W_k, b_k.reshape(1, C), W_v, b_v.reshape(1, NCLS), degp_r)
    q, v, dinv, psum = outs[:4]
    tables0 = outs[4:]

    outp1 = _sc_prop(edges_r, tables0, z2)
    h1, cd1, tables1 = _tc2(outp1, q, dinv)
    outp2 = _sc_prop(edges_r, tables1, z2)

    hid = _tc3(outp2, q, v, h1, cd1, psum, hopwise, teleport)
    return hid[:N]


# speedup vs baseline: 79.6973x; 1.1198x over previous
import functools

import jax
import jax.numpy as jnp
from jax import lax
from jax.experimental import pallas as pl
from jax.experimental.pallas import tpu as pltpu
from jax.experimental.pallas import tpu_sc as plsc

CST = 1e-05

N = 10000
E = 320000
C = 64
NCLS = 8
F = C * NCLS + C
NPH = 3
FP = 192
NPD = 10240
TR = NPD // 16
BLK = 2048
NBLK = NPD // BLK
NW = 32
EW = E // NW
CH = 200
NCH = EW // CH

_SC_PARAMS = pltpu.CompilerParams(use_tc_tiling_on_sc=False)
DT = jnp.bfloat16


def _one_plus_elu(x):
    return jnp.where(x > 0, 1.0 + x, jnp.exp(jnp.minimum(x, 0.0)))


def _mesh():
    return plsc.VectorSubcoreMesh(core_axis_name="c", subcore_axis_name="s")


def _sc_deg_body(edges_ref, z_ref, ones_ref, out_ref, deg_sh, colidx, ones,
                 sem0, sem1):
    cid = lax.axis_index("c")
    sid = lax.axis_index("s")
    wid = sid * 2 + cid
    pltpu.sync_copy(edges_ref.at[1, wid], colidx)
    pltpu.sync_copy(ones_ref, ones)
    pltpu.sync_copy(z_ref, deg_sh.at[pl.ds(sid * TR, TR)])
    plsc.subcore_barrier()
    descs = []
    for c in range(NCH):
        descs.append(pltpu.async_copy(
            ones, deg_sh.at[colidx.at[c]],
            sem0 if c % 2 == 0 else sem1, add=True))
        if c >= 1:
            descs[c - 1].wait()
    descs[NCH - 1].wait()
    plsc.subcore_barrier()
    pltpu.sync_copy(deg_sh.at[pl.ds(sid * TR, TR)],
                    out_ref.at[cid, pl.ds(sid * TR, TR)])


def _sc_deg(edges_r, z16, ones2):
    kfn = functools.partial(
        pl.kernel,
        out_type=jax.ShapeDtypeStruct((2, NPD, 16), jnp.float32),
        mesh=_mesh(),
        scratch_types=[
            pltpu.VMEM_SHARED((NPD, 16), jnp.float32),
            pltpu.VMEM((NCH, CH), jnp.int32),
            pltpu.VMEM((CH, 16), jnp.float32),
            pltpu.SemaphoreType.DMA,
            pltpu.SemaphoreType.DMA,
        ],
        compiler_params=_SC_PARAMS,
    )(_sc_deg_body)
    return kfn(edges_r, z16, ones2)


def _split_tables(xn, trefs):
    xn = xn.astype(DT)
    for p in range(NPH):
        trefs[p][...] = xn[:, p * FP:(p + 1) * FP]


def _tc1_body(x_ref, win_ref, bin_ref, wq_ref, bq_ref, wk_ref, bk_ref,
              wv_ref, bv_ref, degp_ref,
              q_ref, v_ref, dinv_ref, psum_ref,
              t0_ref, t1_ref, t2_ref):
    x = x_ref[...]
    h = jax.nn.relu(jnp.dot(x, win_ref[...], preferred_element_type=jnp.float32)
                    + bin_ref[...])
    q = _one_plus_elu(jnp.dot(h, wq_ref[...], preferred_element_type=jnp.float32)
                      + bq_ref[...])
    kf = _one_plus_elu(jnp.dot(h, wk_ref[...], preferred_element_type=jnp.float32)
                       + bk_ref[...])
    v = jnp.dot(h, wv_ref[...], preferred_element_type=jnp.float32) + bv_ref[...]
    q_ref[...] = q
    v_ref[...] = v
    x0 = jnp.concatenate([kf * v[:, j:j + 1] for j in range(NCLS)] + [kf],
                         axis=-1)
    rows = pl.program_id(0) * BLK + lax.broadcasted_iota(jnp.int32, (BLK, 1), 0)
    x0m = jnp.where(rows < N, x0, 0.0)
    s = jnp.sum(x0m, axis=0, keepdims=True)
    psum_ref[...] = jnp.concatenate([s, jnp.zeros((7, F), jnp.float32)],
                                    axis=0)[None]
    deg = degp_ref[0, 0, :, 0:1] + degp_ref[1, 0, :, 0:1]
    dinv = jnp.where(deg > 0, 1.0 / jnp.maximum(deg, 1e-30), 0.0)
    dinv_ref[...] = dinv
    _split_tables(x0 * dinv, [t0_ref, t1_ref, t2_ref])


def _tc1(x_pad, W_in, b_in, W_q, b_q, W_k, b_k, W_v, b_v, degp_r):
    full = lambda s: pl.BlockSpec(s, lambda i: (0, 0))
    return pl.pallas_call(
        _tc1_body,
        grid=(NBLK,),
        in_specs=[
            pl.BlockSpec((BLK, W_in.shape[0]), lambda i: (i, 0)),
            full(W_in.shape), full((1, C)),
            full(W_q.shape), full((1, C)),
            full(W_k.shape), full((1, C)),
            full(W_v.shape), full((1, NCLS)),
            pl.BlockSpec((2, 1, BLK, 16), lambda i: (0, i, 0, 0)),
        ],
        out_specs=[
            pl.BlockSpec((BLK, C), lambda i: (i, 0)),
            pl.BlockSpec((BLK, NCLS), lambda i: (i, 0)),
            pl.BlockSpec((BLK, 1), lambda i: (i, 0)),
            pl.BlockSpec((1, 8, F), lambda i: (i, 0, 0)),
        ] + [pl.BlockSpec((BLK, FP), lambda i: (i, 0))] * NPH,
        out_shape=[
            jax.ShapeDtypeStruct((NPD, C), jnp.float32),
            jax.ShapeDtypeStruct((NPD, NCLS), jnp.float32),
            jax.ShapeDtypeStruct((NPD, 1), jnp.float32),
            jax.ShapeDtypeStruct((NBLK, 8, F), jnp.float32),
        ] + [jax.ShapeDtypeStruct((NPD, FP), DT)] * NPH,
    )(x_pad, W_in, b_in, W_q, b_q, W_k, b_k, W_v, b_v, degp_r)


def _sc_prop_body(edges_ref, t0, t1, t2, z2_ref, out_ref,
                  xa_sh, rowidx, colidx, buf0, buf1,
                  gs0, gs1, ss0, ss1):
    cid = lax.axis_index("c")
    sid = lax.axis_index("s")
    wid = sid * 2 + cid
    pltpu.sync_copy(edges_ref.at[0, wid], rowidx)
    pltpu.sync_copy(edges_ref.at[1, wid], colidx)
    tables = (t0, t1, t2)
    bufs = (buf0, buf1)
    gsems = (gs0, gs1)
    ssems = (ss0, ss1)
    for p in range(NPH):
        tp = tables[p]
        pltpu.sync_copy(z2_ref, xa_sh.at[pl.ds(sid * TR, TR)])
        plsc.subcore_barrier()
        gd = [None, None]
        sd = [None, None]
        gd[0] = pltpu.async_copy(tp.at[rowidx.at[0]], bufs[0], gsems[0])
        for c in range(NCH):
            b = c % 2
            nb = (c + 1) % 2
            if sd[nb] is not None:
                sd[nb].wait()
                sd[nb] = None
            if c + 1 < NCH:
                gd[nb] = pltpu.async_copy(tp.at[rowidx.at[c + 1]],
                                          bufs[nb], gsems[nb])
            gd[b].wait()
            sd[b] = pltpu.async_copy(bufs[b], xa_sh.at[colidx.at[c]],
                                     ssems[b], add=True)
        for b in range(2):
            if sd[b] is not None:
                sd[b].wait()
        plsc.subcore_barrier()
        pltpu.sync_copy(xa_sh.at[pl.ds(sid * TR, TR)],
                        out_ref.at[cid, p, pl.ds(sid * TR, TR)])
        plsc.subcore_barrier()


def _sc_prop(edges_r, tables, z2):
    kfn = functools.partial(
        pl.kernel,
        out_type=jax.ShapeDtypeStruct((2, NPH, NPD, FP), DT),
        mesh=_mesh(),
        scratch_types=[
            pltpu.VMEM_SHARED((NPD, FP), DT),
            pltpu.VMEM((NCH, CH), jnp.int32),
            pltpu.VMEM((NCH, CH), jnp.int32),
            pltpu.VMEM((CH, FP), DT),
            pltpu.VMEM((CH, FP), DT),
            pltpu.SemaphoreType.DMA,
            pltpu.SemaphoreType.DMA,
            pltpu.SemaphoreType.DMA,
            pltpu.SemaphoreType.DMA,
        ],
        compiler_params=_SC_PARAMS,
    )(_sc_prop_body)
    return kfn(edges_r, *tables, z2)


def _tc2_body(outp_ref, q_ref, dinv_ref, h_ref, cden_ref,
              t0_ref, t1_ref, t2_ref):
    x1 = jnp.concatenate(
        [outp_ref[0, p].astype(jnp.float32) +
         outp_ref[1, p].astype(jnp.float32) for p in range(NPH)], axis=-1)
    q = q_ref[...]
    cols = [jnp.sum(q * x1[:, j * C:(j + 1) * C], axis=1, keepdims=True)
            for j in range(NCLS)]
    h_ref[...] = jnp.concatenate(cols, axis=-1)
    cden_ref[...] = jnp.sum(q * x1[:, NCLS * C:], axis=1, keepdims=True) + CST
    _split_tables(x1 * dinv_ref[...], [t0_ref, t1_ref, t2_ref])


def _tc2(outp, q, dinv):
    outs = pl.pallas_call(
        _tc2_body,
        grid=(NBLK,),
        in_specs=[
            pl.BlockSpec((2, NPH, BLK, FP), lambda i: (0, 0, i, 0)),
            pl.BlockSpec((BLK, C), lambda i: (i, 0)),
            pl.BlockSpec((BLK, 1), lambda i: (i, 0)),
        ],
        out_specs=[pl.BlockSpec((BLK, NCLS), lambda i: (i, 0)),
                   pl.BlockSpec((BLK, 1), lambda i: (i, 0))] +
                  [pl.BlockSpec((BLK, FP), lambda i: (i, 0))] * NPH,
        out_shape=[jax.ShapeDtypeStruct((NPD, NCLS), jnp.float32),
                   jax.ShapeDtypeStruct((NPD, 1), jnp.float32)] +
                  [jax.ShapeDtypeStruct((NPD, FP), DT)] * NPH,
    )(outp, q, dinv)
    return outs[0], outs[1], outs[2:2 + NPH]


def _tc3_body(outp_ref, q_ref, v_ref, h1_ref, cd1_ref, psum_ref,
              hop_ref, tel_ref, hid_ref):
    x2 = jnp.concatenate(
        [outp_ref[0, p].astype(jnp.float32) +
         outp_ref[1, p].astype(jnp.float32) for p in range(NPH)], axis=-1)
    q = q_ref[...]
    cols = [jnp.sum(q * x2[:, j * C:(j + 1) * C], axis=1, keepdims=True)
            for j in range(NCLS)]
    h2 = jnp.concatenate(cols, axis=-1)
    cd2 = jnp.sum(q * x2[:, NCLS * C:], axis=1, keepdims=True) + CST
    tvec = jnp.sum(psum_ref[...], axis=(0, 1)) / float(N)
    tcols = [jnp.dot(q, tvec[j * C:(j + 1) * C][:, None],
                     preferred_element_type=jnp.float32)
             for j in range(NCLS)]
    th = jnp.concatenate(tcols, axis=-1)
    tc = jnp.dot(q, tvec[NCLS * C:][:, None],
                 preferred_element_type=jnp.float32) + CST
    hid_ref[...] = (hop_ref[0] * v_ref[...]
                    + hop_ref[1] * (h1_ref[...] / cd1_ref[...])
                    + hop_ref[2] * (h2 / cd2)
                    + tel_ref[0] * (th / tc))


def _tc3(outp2, q, v, h1, cd1, psum, hopwise, teleport):
    return pl.pallas_call(
        _tc3_body,
        grid=(NBLK,),
        in_specs=[
            pl.BlockSpec((2, NPH, BLK, FP), lambda i: (0, 0, i, 0)),
            pl.BlockSpec((BLK, C), lambda i: (i, 0)),
            pl.BlockSpec((BLK, NCLS), lambda i: (i, 0)),
            pl.BlockSpec((BLK, NCLS), lambda i: (i, 0)),
            pl.BlockSpec((BLK, 1), lambda i: (i, 0)),
            pl.BlockSpec((NBLK, 8, F), lambda i: (0, 0, 0)),
            pl.BlockSpec(memory_space=pltpu.SMEM),
            pl.BlockSpec(memory_space=pltpu.SMEM),
        ],
        out_specs=pl.BlockSpec((BLK, NCLS), lambda i: (i, 0)),
        out_shape=jax.ShapeDtypeStruct((NPD, NCLS), jnp.float32),
    )(outp2, q, v, h1, cd1, psum, hopwise, teleport)


def kernel(node_feat, edge_index, W_in, b_in, W_q, b_q, W_k, b_k, W_v, b_v,
           hopwise, teleport):
    assert node_feat.shape[0] == N and edge_index.shape == (2, E)
    x_pad = jnp.pad(node_feat, ((0, NPD - N), (0, 0)))
    edges_r = edge_index.reshape(2, NW, NCH, CH)

    z2 = jnp.zeros((TR, FP), DT)
    z16 = jnp.zeros((TR, 16), jnp.float32)
    ones2 = jnp.ones((CH, 16), jnp.float32)

    degp = _sc_deg(edges_r, z16, ones2)
    degp_r = degp.reshape(2, NBLK, BLK, 16)

    outs = _tc1(x_pad, W_in, b_in.reshape(1, C), W_q, b_q.reshape(1, C),
                W_k, b_k.reshape(1, C), W_v, b_v.reshape(1, NCLS), degp_r)
    q, v, dinv, psum = outs[:4]
    tables0 = outs[4:]

    outp1 = _sc_prop(edges_r, tables0, z2)
    h1, cd1, tables1 = _tc2(outp1, q, dinv)
    outp2 = _sc_prop(edges_r, tables1, z2)

    hid = _tc3(outp2, q, v, h1, cd1, psum, hopwise, teleport)
    return hid[:N]
